# Initial kernel scaffold; baseline (speedup 1.0000x reference)
import jax
import jax.numpy as jnp
from jax import lax
from jax.experimental import pallas as pl
from jax.experimental.pallas import tpu as pltpu
from jax.experimental.pallas import tpu_sc as plsc

N = 10000
E = 320000
D = 128

NC = 2
NS = 16
L = 16
NW = NC * NS
PER_W = E // NW
C = 80
CHUNKS = PER_W // C
GROUPS = C // L
ROWS_PER_TILE = N // NS


def _prep_body(feat_ref, m1_ref, m2_ref, b_ref, fa_ref, bm_ref):
    feat = feat_ref[...]
    a = jnp.dot(feat, m1_ref[...], preferred_element_type=jnp.float32)
    a = a + b_ref[...][None, :]
    fa_ref[:, :D] = feat
    fa_ref[:, D:] = a
    bm_ref[...] = jnp.dot(feat, m2_ref[...], preferred_element_type=jnp.float32)


def _tc_prep(feat, m1, m2, bias):
    return pl.pallas_call(
        _prep_body,
        out_shape=(
            jax.ShapeDtypeStruct((N, 2 * D), jnp.float32),
            jax.ShapeDtypeStruct((N, D), jnp.float32),
        ),
    )(feat, m1, m2, bias)


def _sc_body(fa_hbm, bm_hbm, src_hbm, dst_hbm, w_hbm, zini_hbm,
             zout_hbm, den_hbm,
             src_v, dst_v, fa_v, b_v, s_v, ex_v, e_buf, w_v, den_l, z_s):
    core = lax.axis_index("c")
    sid = lax.axis_index("s")
    wid = sid * NC + core
    wstart = wid * PER_W

    pltpu.sync_copy(w_hbm, w_v)

    def _zero_den(i, carry):
        den_l[pl.ds(i * L, L)] = jnp.zeros((L,), jnp.float32)
        return carry
    lax.fori_loop(0, N // L, _zero_den, 0)

    pltpu.sync_copy(zini_hbm.at[pl.ds(sid * ROWS_PER_TILE, ROWS_PER_TILE)],
                    z_s.at[pl.ds(sid * ROWS_PER_TILE, ROWS_PER_TILE)])
    plsc.subcore_barrier()

    iota = lax.iota(jnp.int32, L)
    cols = [iota + L * k for k in range(D // L)]

    def _chunk(c, carry):
        base = wstart + c * C
        pltpu.sync_copy(src_hbm.at[pl.ds(base, C)], src_v)
        pltpu.sync_copy(dst_hbm.at[pl.ds(base, C)], dst_v)
        pltpu.sync_copy(fa_hbm.at[src_v], fa_v)
        pltpu.sync_copy(bm_hbm.at[dst_v], b_v)

        def _edge_e(e, carry2):
            fe = jnp.full((L,), e, jnp.int32)
            acc = jnp.zeros((L,), jnp.float32)
            for k in range(D // L):
                av = plsc.load_gather(fa_v, [fe, cols[k] + D])
                bv = plsc.load_gather(b_v, [fe, cols[k]])
                wk = w_v[pl.ds(L * k, L)]
                x = jnp.clip(av + bv, -15.0, 15.0)
                y = jnp.exp(x + x)
                t = 1.0 - 2.0 / (y + 1.0)
                acc = acc + wk * t
            e_buf[e] = jnp.sum(acc)
            return carry2
        lax.fori_loop(0, C, _edge_e, 0)

        for g in range(GROUPS):
            e16 = e_buf[pl.ds(g * L, L)]
            ex16 = jnp.exp(jnp.clip(e16, -80.0, 80.0))
            ex_v[pl.ds(g * L, L)] = ex16
            d16 = dst_v[pl.ds(g * L, L)]
            plsc.addupdate_scatter(den_l, [d16], ex16)

        def _edge_s(e, carry2):
            fe = jnp.full((L,), e, jnp.int32)
            a = plsc.load_gather(ex_v, [fe])
            for k in range(D // L):
                v = plsc.load_gather(fa_v, [fe, cols[k]])
                plsc.store_scatter(s_v, [fe, cols[k]], a * v)
            return carry2
        lax.fori_loop(0, C, _edge_s, 0)

        pltpu.sync_copy(s_v, z_s.at[dst_v], add=True)
        return carry

    lax.fori_loop(0, CHUNKS, _chunk, 0)

    pltpu.sync_copy(den_l, den_hbm.at[wid])
    plsc.subcore_barrier()
    pltpu.sync_copy(z_s.at[pl.ds(sid * ROWS_PER_TILE, ROWS_PER_TILE)],
                    zout_hbm.at[core, pl.ds(sid * ROWS_PER_TILE, ROWS_PER_TILE)])


def _sc_main(fa, bm, src, dst, w, zini):
    f = pl.kernel(
        _sc_body,
        out_type=(
            jax.ShapeDtypeStruct((NC, N, D), jnp.float32),
            jax.ShapeDtypeStruct((NW, N), jnp.float32),
        ),
        mesh=plsc.VectorSubcoreMesh(core_axis_name="c", subcore_axis_name="s"),
        scratch_types=[
            pltpu.VMEM((C,), jnp.int32),
            pltpu.VMEM((C,), jnp.int32),
            pltpu.VMEM((C, 2 * D), jnp.float32),
            pltpu.VMEM((C, D), jnp.float32),
            pltpu.VMEM((C, D), jnp.float32),
            pltpu.VMEM((C,), jnp.float32),
            pltpu.VMEM((C,), jnp.float32),
            pltpu.VMEM((D,), jnp.float32),
            pltpu.VMEM((N,), jnp.float32),
            pltpu.VMEM_SHARED((N, D), jnp.float32),
        ],
    )
    return f(fa, bm, src, dst, w, zini)


def _fin_body(z2_ref, den_ref, out_ref):
    zsum = z2_ref[0] + z2_ref[1]
    den = jnp.sum(den_ref[...], axis=0)
    safe = den > 0.0
    deninv = jnp.where(safe, 1.0 / jnp.where(safe, den, 1.0), 0.0)
    out_ref[...] = zsum * deninv[:, None]


def _tc_finish(z2, den):
    return pl.pallas_call(
        _fin_body,
        out_shape=jax.ShapeDtypeStruct((N, D), jnp.float32),
    )(z2, den)


@jax.jit
def kernel(feat, edge_index, attn_fc_w, attn_fc_b, attn_out_w):
    src = edge_index[0]
    dst = edge_index[1]
    wt = attn_fc_w.T
    m1 = wt[:D, :]
    m2 = wt[D:, :]
    w = attn_out_w[0]
    fa, bm = _tc_prep(feat, m1, m2, attn_fc_b)
    zini = jnp.zeros((N, D), jnp.float32)
    z2, den = _sc_main(fa, bm, src, dst, w, zini)
    return _tc_finish(z2, den)


# speedup vs baseline: 6.5594x; 6.5594x over previous
import jax
import jax.numpy as jnp
from jax import lax
from jax.experimental import pallas as pl
from jax.experimental.pallas import tpu as pltpu
from jax.experimental.pallas import tpu_sc as plsc

N = 10000
E = 320000
D = 128

NC = 2
NS = 16
L = 16
NW = NC * NS
PER_W = E // NW
C = 80
CHUNKS = PER_W // C
GROUPS = C // L
RPT = 624
TAIL = N - RPT * NS


def _prep_body(feat_ref, m1_ref, m2_ref, b_ref, fa_ref, bm_ref):
    feat = feat_ref[...]
    a = jnp.dot(feat, m1_ref[...], preferred_element_type=jnp.float32)
    a = a + b_ref[...][None, :]
    fa_ref[:, :D] = feat
    fa_ref[:, D:] = a
    bm_ref[...] = jnp.dot(feat, m2_ref[...], preferred_element_type=jnp.float32)


def _tc_prep(feat, m1, m2, bias):
    return pl.pallas_call(
        _prep_body,
        out_shape=(
            jax.ShapeDtypeStruct((N, 2 * D), jnp.float32),
            jax.ShapeDtypeStruct((N, D), jnp.float32),
        ),
    )(feat, m1, m2, bias)


def _sc_body(fa_hbm, bm_hbm, src_hbm, dst_hbm, w_hbm, zini_hbm,
             zout_hbm, den_hbm,
             src_v, dst_v, fa_v, b_v, ex_v, w_v, den_l, z_s):
    core = lax.axis_index("c")
    sid = lax.axis_index("s")
    wid = sid * NC + core
    wstart = wid * PER_W

    pltpu.sync_copy(w_hbm, w_v)

    def _zero_den(i, carry):
        den_l[pl.ds(i * L, L)] = jnp.zeros((L,), jnp.float32)
        return carry
    lax.fori_loop(0, N // L, _zero_den, 0)

    pltpu.sync_copy(zini_hbm.at[pl.ds(sid * RPT, RPT)],
                    z_s.at[pl.ds(sid * RPT, RPT)])

    @pl.when(sid == NS - 1)
    def _():
        pltpu.sync_copy(zini_hbm.at[pl.ds(RPT * NS, TAIL)],
                        z_s.at[pl.ds(RPT * NS, TAIL)])
    plsc.subcore_barrier()

    iota = lax.iota(jnp.int32, L)
    cols = [iota + L * k for k in range(D // L)]

    def _chunk(c, carry):
        base = wstart + c * C
        pltpu.sync_copy(src_hbm.at[pl.ds(base, C)], src_v)
        pltpu.sync_copy(dst_hbm.at[pl.ds(base, C)], dst_v)
        pltpu.sync_copy(fa_hbm.at[src_v], fa_v)
        pltpu.sync_copy(bm_hbm.at[dst_v], b_v)

        for g in range(GROUPS):
            def _edge_e(j, evec):
                fe = jnp.full((L,), g * L + j, jnp.int32)
                acc = jnp.zeros((L,), jnp.float32)
                for k in range(D // L):
                    av = plsc.load_gather(fa_v, [fe, cols[k] + D])
                    bv = plsc.load_gather(b_v, [fe, cols[k]])
                    wk = w_v[pl.ds(L * k, L)]
                    x = jnp.clip(av + bv, -15.0, 15.0)
                    y = jnp.exp(x + x)
                    t = 1.0 - 2.0 / (y + 1.0)
                    acc = acc + wk * t
                return jnp.where(iota == j, jnp.sum(acc), evec)
            e16 = lax.fori_loop(0, L, _edge_e, jnp.zeros((L,), jnp.float32))
            ex16 = jnp.exp(jnp.clip(e16, -80.0, 80.0))
            ex_v[pl.ds(g * L, L)] = ex16
            d16 = dst_v[pl.ds(g * L, L)]
            plsc.addupdate_scatter(den_l, [d16], ex16)

        def _edge_s(e, carry2):
            fe = jnp.full((L,), e, jnp.int32)
            a = plsc.load_gather(ex_v, [fe])
            for k in range(D // L):
                v = plsc.load_gather(fa_v, [fe, cols[k]])
                plsc.store_scatter(b_v, [fe, cols[k]], a * v)
            return carry2
        lax.fori_loop(0, C, _edge_s, 0)

        pltpu.sync_copy(b_v, z_s.at[dst_v], add=True)
        return carry

    lax.fori_loop(0, CHUNKS, _chunk, 0)

    pltpu.sync_copy(den_l, den_hbm.at[pl.ds(wid * N, N)])
    plsc.subcore_barrier()
    pltpu.sync_copy(z_s.at[pl.ds(sid * RPT, RPT)],
                    zout_hbm.at[core, pl.ds(sid * RPT, RPT)])

    @pl.when(sid == NS - 1)
    def _():
        pltpu.sync_copy(z_s.at[pl.ds(RPT * NS, TAIL)],
                        zout_hbm.at[core, pl.ds(RPT * NS, TAIL)])


def _sc_main(fa, bm, src, dst, w, zini):
    f = pl.kernel(
        _sc_body,
        out_type=(
            jax.ShapeDtypeStruct((NC, N, D), jnp.float32),
            jax.ShapeDtypeStruct((NW * N,), jnp.float32),
        ),
        mesh=plsc.VectorSubcoreMesh(core_axis_name="c", subcore_axis_name="s"),
        compiler_params=pltpu.CompilerParams(needs_layout_passes=False),
        scratch_types=[
            pltpu.VMEM((C,), jnp.int32),
            pltpu.VMEM((C,), jnp.int32),
            pltpu.VMEM((C, 2 * D), jnp.float32),
            pltpu.VMEM((C, D), jnp.float32),
            pltpu.VMEM((C,), jnp.float32),
            pltpu.VMEM((D,), jnp.float32),
            pltpu.VMEM((N,), jnp.float32),
            pltpu.VMEM_SHARED((N, D), jnp.float32),
        ],
    )
    return f(fa, bm, src, dst, w, zini)


def _fin_body(z2_ref, den_ref, out_ref):
    zsum = z2_ref[0] + z2_ref[1]
    den = jnp.sum(den_ref[...], axis=0)
    safe = den > 0.0
    deninv = jnp.where(safe, 1.0 / jnp.where(safe, den, 1.0), 0.0)
    out_ref[...] = zsum * deninv[:, None]


def _tc_finish(z2, den):
    return pl.pallas_call(
        _fin_body,
        out_shape=jax.ShapeDtypeStruct((N, D), jnp.float32),
    )(z2, den)


@jax.jit
def kernel(feat, edge_index, attn_fc_w, attn_fc_b, attn_out_w):
    src = edge_index[0]
    dst = edge_index[1]
    wt = attn_fc_w.T
    m1 = wt[:D, :]
    m2 = wt[D:, :]
    w = attn_out_w[0]
    fa, bm = _tc_prep(feat, m1, m2, attn_fc_b)
    zini = jnp.zeros((N, D), jnp.float32)
    z2, den = _sc_main(fa, bm, src, dst, w, zini)
    return _tc_finish(z2, den.reshape(NW, N))
